# baseline (speedup 1.0000x reference)
import functools

import jax
import jax.numpy as jnp
from jax import lax
from jax.experimental import pallas as pl
from jax.experimental.pallas import tpu as pltpu
from jax.experimental.pallas import tpu_sc as plsc

_CHUNK = 128
_WU = 32
_WM = 16


@functools.cache
def _make_kernel(B, D_u, D_m, V_u, V_m):
    info = plsc.get_sparse_core_info()
    NC, NS = info.num_cores, info.num_subcores
    NW = NC * NS
    assert B % (NW * _CHUNK) == 0
    b_per_w = B // NW
    n_ch = b_per_w // _CHUNK
    D = D_u + D_m
    mesh = plsc.VectorSubcoreMesh(core_axis_name="c", subcore_axis_name="s")

    @functools.partial(
        pl.kernel,
        mesh=mesh,
        out_type=jax.ShapeDtypeStruct((B, D), jnp.float32),
        compiler_params=pltpu.CompilerParams(use_tc_tiling_on_sc=False),
        scratch_types=[
            pltpu.VMEM((n_ch, _CHUNK), jnp.int32),
            pltpu.VMEM((n_ch, _CHUNK), jnp.int32),
            pltpu.VMEM((n_ch, _CHUNK), jnp.int32),
            pltpu.VMEM((n_ch, _CHUNK), jnp.int32),
            pltpu.VMEM((n_ch, _CHUNK), jnp.int32),
            pltpu.VMEM((n_ch, _CHUNK), jnp.int32),
            pltpu.VMEM((b_per_w,), jnp.int32),
            pltpu.VMEM((b_per_w,), jnp.int32),
            pltpu.VMEM((_CHUNK, _WU), jnp.float32),
            pltpu.VMEM((_CHUNK, _WU), jnp.float32),
            pltpu.VMEM((_CHUNK, _WM), jnp.float32),
            pltpu.VMEM((_CHUNK, _WM), jnp.float32),
            pltpu.VMEM((_CHUNK, 2 * _WU), jnp.float32),
            pltpu.VMEM((_CHUNK, 2 * _WM), jnp.float32),
            pltpu.VMEM((b_per_w, D), jnp.float32),
            pltpu.SemaphoreType.DMA,
        ],
    )
    def k(uid_hbm, mid_hbm, ut_hbm, mt_hbm, out_hbm,
          uids, mids, ga0, ga1, gb0, gb1, offu, offm,
          tu0, tu1, tm0, tm1, duw, dmw, comb, sem):
        wid = lax.axis_index("s") * NC + lax.axis_index("c")
        base = wid * b_per_w

        for c in range(n_ch):
            pltpu.sync_copy(uid_hbm.at[pl.ds(base + c * _CHUNK, _CHUNK)],
                            uids.at[c])
            pltpu.sync_copy(mid_hbm.at[pl.ds(base + c * _CHUNK, _CHUNK)],
                            mids.at[c])

        for c in range(n_ch):
            for g in range(_CHUNK // 16):
                s = pl.ds(g * 16, 16)
                fs = pl.ds(c * _CHUNK + g * 16, 16)
                wu = uids[c, s] * D_u
                a0 = lax.shift_right_logical(wu, 5)
                ga0[c, s] = a0
                ga1[c, s] = jnp.minimum(a0 + 1, V_u - 1)
                offu[fs] = jnp.bitwise_and(wu, _WU - 1)
                wm = mids[c, s] * D_m
                b0 = lax.shift_right_logical(wm, 4)
                gb0[c, s] = b0
                gb1[c, s] = jnp.minimum(b0 + 1, V_m - 1)
                offm[fs] = jnp.bitwise_and(wm, _WM - 1)

        for c in range(n_ch):
            cps = [
                pltpu.async_copy(ut_hbm.at[ga0.at[c]], tu0, sem),
                pltpu.async_copy(ut_hbm.at[ga1.at[c]], tu1, sem),
                pltpu.async_copy(mt_hbm.at[gb0.at[c]], tm0, sem),
                pltpu.async_copy(mt_hbm.at[gb1.at[c]], tm1, sem),
            ]
            for cp in cps:
                cp.wait()

            @pl.loop(0, _CHUNK // 16)
            def _merge(g):
                r0 = g * 16
                ovu = offu[pl.ds(c * _CHUNK + r0, 16)]
                ovm = offm[pl.ds(c * _CHUNK + r0, 16)]
                for l in range(16):
                    rr = r0 + l
                    r = c * _CHUNK + rr
                    duw[rr, pl.ds(0, 16)] = tu0[rr, pl.ds(0, 16)]
                    duw[rr, pl.ds(16, 16)] = tu0[rr, pl.ds(16, 16)]
                    duw[rr, pl.ds(32, 16)] = tu1[rr, pl.ds(0, 16)]
                    duw[rr, pl.ds(48, 16)] = tu1[rr, pl.ds(16, 16)]
                    dmw[rr, pl.ds(0, 16)] = tm0[rr, pl.ds(0, 16)]
                    dmw[rr, pl.ds(16, 16)] = tm1[rr, pl.ds(0, 16)]
                    o = ovu[l]
                    p = ovm[l]
                    comb[r, pl.ds(0, 16)] = duw[rr, pl.ds(o, 16)]
                    comb[r, pl.ds(15, 16)] = duw[rr, pl.ds(o + 15, 16)]
                    comb[r, pl.ds(D_u, 16)] = dmw[rr, pl.ds(p, 16)]
                    comb[r, pl.ds(D_u + 1, 16)] = dmw[rr, pl.ds(p + 1, 16)]

        pltpu.sync_copy(comb, out_hbm.at[pl.ds(base, b_per_w)])

    return k


def kernel(user_ids, movie_ids, user_table, movie_table):
    B = user_ids.shape[0]
    N_u, D_u = user_table.shape
    N_m, D_m = movie_table.shape
    ut32 = user_table.reshape(N_u * D_u // _WU, _WU)
    mt16 = movie_table.reshape(N_m * D_m // _WM, _WM)
    k = _make_kernel(B, D_u, D_m, ut32.shape[0], mt16.shape[0])
    return k(user_ids, movie_ids, ut32, mt16)


# speedup vs baseline: 1.1454x; 1.1454x over previous
import functools

import jax
import jax.numpy as jnp
from jax import lax
from jax.experimental import pallas as pl
from jax.experimental.pallas import tpu as pltpu
from jax.experimental.pallas import tpu_sc as plsc

_CHUNK = 128


@functools.cache
def _make_kernel(B, D_u, D_m):
    info = plsc.get_sparse_core_info()
    NC, NS = info.num_cores, info.num_subcores
    NW = NC * NS
    assert B % (NW * _CHUNK) == 0
    b_per_w = B // NW
    n_ch = b_per_w // _CHUNK
    D = D_u + D_m
    mesh = plsc.VectorSubcoreMesh(core_axis_name="c", subcore_axis_name="s")

    @functools.partial(
        pl.kernel,
        mesh=mesh,
        out_type=jax.ShapeDtypeStruct((B, D), jnp.float32),
        compiler_params=pltpu.CompilerParams(use_tc_tiling_on_sc=True),
        scratch_types=[
            pltpu.VMEM((n_ch, _CHUNK), jnp.int32),
            pltpu.VMEM((n_ch, _CHUNK), jnp.int32),
            pltpu.VMEM((_CHUNK, 128), jnp.float32),
            pltpu.VMEM((_CHUNK, 128), jnp.float32),
            pltpu.VMEM((b_per_w, D), jnp.float32),
            pltpu.SemaphoreType.DMA,
        ],
    )
    def k(uid_hbm, mid_hbm, ut_hbm, mt_hbm, out_hbm,
          uids, mids, tu, tm, comb, sem):
        wid = lax.axis_index("s") * NC + lax.axis_index("c")
        base = wid * b_per_w
        for c in range(n_ch):
            pltpu.sync_copy(uid_hbm.at[pl.ds(base + c * _CHUNK, _CHUNK)],
                            uids.at[c])
            pltpu.sync_copy(mid_hbm.at[pl.ds(base + c * _CHUNK, _CHUNK)],
                            mids.at[c])
        for c in range(n_ch):
            cu = pltpu.async_copy(ut_hbm.at[uids.at[c]], tu, sem)
            cm = pltpu.async_copy(mt_hbm.at[mids.at[c]], tm, sem)
            cu.wait()
            cm.wait()

            @pl.loop(0, _CHUNK)
            def _merge(rr):
                r = c * _CHUNK + rr
                comb[r, pl.ds(0, 16)] = tu[rr, pl.ds(0, 16)]
                comb[r, pl.ds(15, 16)] = tu[rr, pl.ds(15, 16)]
                comb[r, pl.ds(D_u, 16)] = tm[rr, pl.ds(0, 16)]
                comb[r, pl.ds(D_u + 1, 16)] = tm[rr, pl.ds(1, 16)]

        pltpu.sync_copy(comb, out_hbm.at[pl.ds(base, b_per_w)])

    return k


def kernel(user_ids, movie_ids, user_table, movie_table):
    B = user_ids.shape[0]
    N_u, D_u = user_table.shape
    N_m, D_m = movie_table.shape
    ut128 = jnp.pad(user_table, ((0, 0), (0, 128 - D_u)))
    mt128 = jnp.pad(movie_table, ((0, 0), (0, 128 - D_m)))
    k = _make_kernel(B, D_u, D_m)
    return k(user_ids, movie_ids, ut128, mt128)


# speedup vs baseline: 1.1459x; 1.0004x over previous
import functools

import jax
import jax.numpy as jnp
from jax import lax
from jax.experimental import pallas as pl
from jax.experimental.pallas import tpu as pltpu
from jax.experimental.pallas import tpu_sc as plsc

_CHUNK = 128
_W = 128


@functools.cache
def _make_kernel(B, D_u, D_m):
    info = plsc.get_sparse_core_info()
    NC, NS = info.num_cores, info.num_subcores
    NW = NC * NS
    assert B % (NW * _CHUNK) == 0
    b_per_w = B // NW
    n_ch = b_per_w // _CHUNK
    D = D_u + D_m
    mesh = plsc.VectorSubcoreMesh(core_axis_name="c", subcore_axis_name="s")

    @functools.partial(
        pl.kernel,
        mesh=mesh,
        out_type=jax.ShapeDtypeStruct((B, D), jnp.float32),
        compiler_params=pltpu.CompilerParams(use_tc_tiling_on_sc=True),
        scratch_types=[
            pltpu.VMEM((n_ch, _CHUNK), jnp.int32),
            pltpu.VMEM((n_ch, _CHUNK), jnp.int32),
            pltpu.VMEM((_CHUNK, _W), jnp.float32),
            pltpu.VMEM((_CHUNK, _W), jnp.float32),
            pltpu.VMEM((b_per_w, D), jnp.float32),
            pltpu.SemaphoreType.DMA,
        ],
    )
    def k(uid_hbm, mid_hbm, ut_hbm, mt_hbm, out_hbm,
          uids, mids, tu, tm, comb, sem):
        wid = lax.axis_index("s") * NC + lax.axis_index("c")
        base = wid * b_per_w
        for c in range(n_ch):
            pltpu.sync_copy(uid_hbm.at[pl.ds(base + c * _CHUNK, _CHUNK)],
                            uids.at[c])
            pltpu.sync_copy(mid_hbm.at[pl.ds(base + c * _CHUNK, _CHUNK)],
                            mids.at[c])
        for c in range(n_ch):
            cu = pltpu.async_copy(ut_hbm.at[uids.at[c]], tu, sem)
            cm = pltpu.async_copy(mt_hbm.at[mids.at[c]], tm, sem)
            cu.wait()
            cm.wait()

            @pl.loop(0, _CHUNK)
            def _merge(rr):
                r = c * _CHUNK + rr
                comb[r, pl.ds(0, 16)] = tu[rr, pl.ds(0, 16)]
                comb[r, pl.ds(15, 16)] = tu[rr, pl.ds(15, 16)]
                comb[r, pl.ds(D_u, 16)] = tm[rr, pl.ds(0, 16)]
                comb[r, pl.ds(D_u + 1, 16)] = tm[rr, pl.ds(1, 16)]

        pltpu.sync_copy(comb, out_hbm.at[pl.ds(base, b_per_w)])

    return k


def kernel(user_ids, movie_ids, user_table, movie_table):
    B = user_ids.shape[0]
    N_u, D_u = user_table.shape
    N_m, D_m = movie_table.shape
    ut128 = jnp.pad(user_table, ((0, 0), (0, _W - D_u)))
    mt128 = jnp.pad(movie_table, ((0, 0), (0, _W - D_m)))
    k = _make_kernel(B, D_u, D_m)
    return k(user_ids, movie_ids, ut128, mt128)
